# baseline (speedup 1.0000x reference)
import functools

import numpy as np

import jax
import jax.numpy as jnp
from jax import lax
from jax.experimental import pallas as pl
from jax.experimental.pallas import tpu as pltpu
from jax.experimental.pallas import tpu_sc as plsc

CHUNK = 128
NBUF = 4


def _diag_tables(embed: int):
  l = np.arange(16, dtype=np.int32)
  nj = CHUNK // 16
  ne = embed // 16
  rowt = l[None, :] + 16 * np.arange(nj, dtype=np.int32)[:, None]
  colt = np.empty((16, ne, 16), dtype=np.int32)
  for d in range(16):
    colt[d] = (l + d) % 16 + 16 * np.arange(ne, dtype=np.int32)[:, None]
  return rowt, colt


@functools.lru_cache(maxsize=None)
def _make_gather(vocab: int, embed: int, batch: int, hist: int):
  info = plsc.get_sparse_core_info()
  nw = info.num_cores * info.num_subcores
  nc = info.num_cores

  bpw = batch // nw
  nck = bpw // CHUNK
  assert batch == nw * nck * CHUNK and nck == NBUF and hist % 2 == 0
  nblk = (CHUNK // 16) * (embed // 16)

  mesh = plsc.VectorSubcoreMesh(core_axis_name="c", subcore_axis_name="s")

  @functools.partial(
      pl.kernel,
      mesh=mesh,
      compiler_params=pltpu.CompilerParams(use_tc_tiling_on_sc=False,
                                           needs_layout_passes=False),
      out_type=jax.ShapeDtypeStruct((hist, embed, batch), jnp.float32),
      scratch_types=[
          pltpu.VMEM((2, NBUF, CHUNK), jnp.int32),
          pltpu.VMEM((2, NBUF, CHUNK, embed), jnp.float32),
          pltpu.VMEM((NBUF, embed, CHUNK), jnp.float32),
          pltpu.VMEM((CHUNK // 16, 16), jnp.int32),
          pltpu.VMEM((16, embed // 16, 16), jnp.int32),
          pltpu.SemaphoreType.DMA((2, NBUF)),
          pltpu.SemaphoreType.DMA((NBUF,)),
          pltpu.SemaphoreType.DMA((2,)),
      ],
  )
  def gather_kernel(idx_hbm, table_hbm, rowt_hbm, colt_hbm, out_hbm, idx_v,
                    rows_v, t_v, rowt_v, colt_v, gsem, wsem, isem):
    wid = lax.axis_index("s") * nc + lax.axis_index("c")
    b_base = wid * bpw

    pltpu.sync_copy(rowt_hbm, rowt_v)
    pltpu.sync_copy(colt_hbm, colt_v)
    rvs = [rowt_v[j, :] for j in range(CHUNK // 16)]

    pltpu.async_copy(idx_hbm.at[wid, 0], idx_v.at[0], isem.at[0])
    pltpu.async_copy(idx_hbm.at[wid, 1], idx_v.at[1], isem.at[1])

    def issue_group(h, s):
      pltpu.make_async_copy(idx_hbm.at[wid, h], idx_v.at[s],
                            isem.at[s]).wait()
      for b in range(NBUF):
        pltpu.async_copy(table_hbm.at[idx_v.at[s, b]], rows_v.at[s, b],
                         gsem.at[s, b])

    def drain_group(h, s):
      for b in range(NBUF):
        pltpu.make_async_copy(table_hbm.at[idx_v.at[s, b]], rows_v.at[s, b],
                              gsem.at[s, b]).wait()
      if isinstance(h, int):
        if h + 2 < hist:
          pltpu.async_copy(idx_hbm.at[wid, h + 2], idx_v.at[s], isem.at[s])
      else:
        @pl.when(h + 2 < hist)
        def _():
          pltpu.async_copy(idx_hbm.at[wid, h + 2], idx_v.at[s], isem.at[s])
      for b in range(NBUF):
        @pl.when(h >= 1)
        def _():
          pltpu.make_async_copy(
              t_v.at[b], out_hbm.at[h, :, pl.ds(b_base + b * CHUNK, CHUNK)],
              wsem.at[b]).wait()
        r_t = rows_v.at[s, b]
        t_t = t_v.at[b]

        def transpose_diag(d):
          cvs = [colt_v[d, e, :] for e in range(embed // 16)]
          for j in range(0, CHUNK // 16, 4):
            vals = [plsc.load_gather(r_t, [rvs[j + i], cvs[e]])
                    for i in range(4) for e in range(embed // 16)]
            for i in range(4):
              for e in range(embed // 16):
                plsc.store_scatter(t_t, [cvs[e], rvs[j + i]],
                                   vals[i * (embed // 16) + e])

        pl.loop(0, 16)(transpose_diag)
        pltpu.async_copy(
            t_v.at[b], out_hbm.at[h, :, pl.ds(b_base + b * CHUNK, CHUNK)],
            wsem.at[b])

    issue_group(0, 0)

    def body(hh):
      h = hh * 2
      issue_group(h + 1, 1)
      drain_group(h, 0)
      issue_group(h + 2, 0)
      drain_group(h + 1, 1)

    pl.loop(0, hist // 2 - 1)(body)

    h_last = hist - 2
    issue_group(h_last + 1, 1)
    drain_group(h_last, 0)
    drain_group(h_last + 1, 1)

    for b in range(NBUF):
      pltpu.make_async_copy(
          t_v.at[b],
          out_hbm.at[hist - 1, :, pl.ds(b_base + b * CHUNK, CHUNK)],
          wsem.at[b]).wait()

  return gather_kernel, nw


@jax.jit
def kernel(tensor, table):
  batch, hist = tensor.shape
  vocab, embed = table.shape
  fn, nw = _make_gather(vocab, embed, batch, hist)
  idx = tensor.T.reshape(hist, nw, NBUF, CHUNK).transpose(1, 0, 2, 3)
  rowt, colt = _diag_tables(embed)
  out = fn(idx, table, jnp.asarray(rowt), jnp.asarray(colt))
  return out.transpose(2, 0, 1)


# speedup vs baseline: 1.0065x; 1.0065x over previous
import functools

import numpy as np

import jax
import jax.numpy as jnp
from jax import lax
from jax.experimental import pallas as pl
from jax.experimental.pallas import tpu as pltpu
from jax.experimental.pallas import tpu_sc as plsc

CHUNK = 128
NBUF = 4


def _diag_tables(embed: int):
  l = np.arange(16, dtype=np.int32)
  nj = CHUNK // 16
  ne = embed // 16
  rowt = l[None, :] + 16 * np.arange(nj, dtype=np.int32)[:, None]
  colt = np.empty((16, ne, 16), dtype=np.int32)
  for d in range(16):
    colt[d] = (l + d) % 16 + 16 * np.arange(ne, dtype=np.int32)[:, None]
  return rowt, colt


@functools.lru_cache(maxsize=None)
def _make_gather(vocab: int, embed: int, batch: int, hist: int):
  info = plsc.get_sparse_core_info()
  nw = info.num_cores * info.num_subcores
  nc = info.num_cores

  bpw = batch // nw
  nck = bpw // CHUNK
  assert batch == nw * nck * CHUNK and nck == NBUF and hist % 2 == 0
  nblk = (CHUNK // 16) * (embed // 16)

  mesh = plsc.VectorSubcoreMesh(core_axis_name="c", subcore_axis_name="s")

  @functools.partial(
      pl.kernel,
      mesh=mesh,
      compiler_params=pltpu.CompilerParams(use_tc_tiling_on_sc=False,
                                           needs_layout_passes=False),
      out_type=jax.ShapeDtypeStruct((hist, embed, batch), jnp.float32),
      scratch_types=[
          pltpu.VMEM((2, NBUF, CHUNK), jnp.int32),
          pltpu.VMEM((2, NBUF, CHUNK, embed), jnp.float32),
          pltpu.VMEM((NBUF, embed, CHUNK), jnp.float32),
          pltpu.VMEM((CHUNK // 16, 16), jnp.int32),
          pltpu.VMEM((16, embed // 16, 16), jnp.int32),
          pltpu.SemaphoreType.DMA((2, NBUF)),
          pltpu.SemaphoreType.DMA((NBUF,)),
          pltpu.SemaphoreType.DMA((2,)),
      ],
  )
  def gather_kernel(idx_hbm, table_hbm, rowt_hbm, colt_hbm, out_hbm, idx_v,
                    rows_v, t_v, rowt_v, colt_v, gsem, wsem, isem):
    wid = lax.axis_index("s") * nc + lax.axis_index("c")
    b_base = wid * bpw

    pltpu.sync_copy(rowt_hbm, rowt_v)
    pltpu.sync_copy(colt_hbm, colt_v)
    rvs = [rowt_v[j, :] for j in range(CHUNK // 16)]

    pltpu.async_copy(idx_hbm.at[wid, 0], idx_v.at[0], isem.at[0])
    pltpu.async_copy(idx_hbm.at[wid, 1], idx_v.at[1], isem.at[1])

    def issue_group(h, s):
      pltpu.make_async_copy(idx_hbm.at[wid, h], idx_v.at[s],
                            isem.at[s]).wait()
      for b in range(NBUF):
        pltpu.async_copy(table_hbm.at[idx_v.at[s, b]], rows_v.at[s, b],
                         gsem.at[s, b])

    def drain_group(h, s):
      for b in range(NBUF):
        pltpu.make_async_copy(table_hbm.at[idx_v.at[s, b]], rows_v.at[s, b],
                              gsem.at[s, b]).wait()
      if isinstance(h, int):
        if h + 2 < hist:
          pltpu.async_copy(idx_hbm.at[wid, h + 2], idx_v.at[s], isem.at[s])
      else:
        @pl.when(h + 2 < hist)
        def _():
          pltpu.async_copy(idx_hbm.at[wid, h + 2], idx_v.at[s], isem.at[s])
      for b in range(NBUF):
        @pl.when(h >= 1)
        def _():
          pltpu.make_async_copy(
              t_v.at[b], out_hbm.at[h, :, pl.ds(b_base + b * CHUNK, CHUNK)],
              wsem.at[b]).wait()
        r_t = rows_v.at[s, b]
        t_t = t_v.at[b]

        def transpose_diag(d):
          cvs = [colt_v[d, e, :] for e in range(embed // 16)]
          for j in range(0, CHUNK // 16, 2):
            vals = [plsc.load_gather(r_t, [rvs[j + i], cvs[e]])
                    for i in range(2) for e in range(embed // 16)]
            for i in range(2):
              for e in range(embed // 16):
                plsc.store_scatter(t_t, [cvs[e], rvs[j + i]],
                                   vals[i * (embed // 16) + e])

        pl.loop(0, 16)(transpose_diag)
        pltpu.async_copy(
            t_v.at[b], out_hbm.at[h, :, pl.ds(b_base + b * CHUNK, CHUNK)],
            wsem.at[b])

    issue_group(0, 0)

    def body(hh):
      h = hh * 2
      issue_group(h + 1, 1)
      drain_group(h, 0)
      issue_group(h + 2, 0)
      drain_group(h + 1, 1)

    pl.loop(0, hist // 2 - 1)(body)

    h_last = hist - 2
    issue_group(h_last + 1, 1)
    drain_group(h_last, 0)
    drain_group(h_last + 1, 1)

    for b in range(NBUF):
      pltpu.make_async_copy(
          t_v.at[b],
          out_hbm.at[hist - 1, :, pl.ds(b_base + b * CHUNK, CHUNK)],
          wsem.at[b]).wait()

  return gather_kernel, nw


@jax.jit
def kernel(tensor, table):
  batch, hist = tensor.shape
  vocab, embed = table.shape
  fn, nw = _make_gather(vocab, embed, batch, hist)
  idx = tensor.T.reshape(hist, nw, NBUF, CHUNK).transpose(1, 0, 2, 3)
  rowt, colt = _diag_tables(embed)
  out = fn(idx, table, jnp.asarray(rowt), jnp.asarray(colt))
  return out.transpose(2, 0, 1)
